# baseline (speedup 1.0000x reference)
import functools
import numpy as np
import jax
import jax.numpy as jnp
from jax import lax
from jax.experimental import pallas as pl
from jax.experimental.pallas import tpu as pltpu
from jax.experimental.pallas import tpu_sc as plsc

_H = 128
_N = 10000
_E = 320000
_B = 32
_L = 2

_NP = 10240
_NC = 2
_NS = 16
_NW = _NC * _NS
_CH = 128
_NCH = 80
_EPW = _CH * _NCH
_EP = _EPW * _NW
_RT = _NP // _NS

_f32 = jnp.float32


def _silu(v):
    return v * jax.nn.sigmoid(v)


def _pre_body(x_ref, p16_ref, batch_ref, t_ref,
              W_emb, b_emb, Wt1, bt1, Wt2, bt2, Wtp, btp,
              Wa, Wb, eb1l,
              h_out, pc_out, posp_out, ta_out, tb_out):
    p16 = p16_ref[...]
    batch = batch_ref[...]
    onehot = (batch == lax.broadcasted_iota(jnp.int32, (_NP, _B), 1)
              ).astype(_f32)
    sums = lax.dot_general(onehot, p16, (((0,), (0,)), ((), ())))
    cnts = jnp.sum(onehot, axis=0, keepdims=True)
    mean = sums / jnp.maximum(cnts.T, 1.0)
    pos_c = p16 - jnp.dot(onehot, mean)

    tf = t_ref[...].astype(_f32)
    half = _H // 2
    freqs = jnp.exp(lax.broadcasted_iota(jnp.int32, (1, half), 1).astype(_f32)
                    * (-np.log(10000.0) / half))
    args = tf * freqs
    temb = jnp.concatenate([jnp.sin(args), jnp.cos(args)], axis=-1)
    temb = _silu(jnp.dot(temb, Wt1[...]) + bt1[...])
    temb = jnp.dot(temb, Wt2[...]) + bt2[...]
    tn = jnp.dot(_silu(temb), Wtp[...]) + btp[...]

    h = jnp.dot(x_ref[...], W_emb[...]) + b_emb[...] + jnp.dot(onehot, tn)
    h_out[...] = h
    pc_out[...] = pos_c
    posp_out[...] = jnp.transpose(pos_c)
    ta_out[...] = jnp.dot(h, Wa[...]).astype(jnp.bfloat16)
    tb_out[...] = (jnp.dot(h, Wb[...]) + eb1l[...]).astype(jnp.bfloat16)


def _run_pre(x_p, p16, batch_p, t2, W_emb, b_emb, Wt1, bt1, Wt2, bt2,
             Wtp, btp, Wa, Wb, eb1l):
    out_shape = [jax.ShapeDtypeStruct((_NP, _H), _f32),
                 jax.ShapeDtypeStruct((_NP, 16), _f32),
                 jax.ShapeDtypeStruct((16, _NP), _f32),
                 jax.ShapeDtypeStruct((_NP, _H), jnp.bfloat16),
                 jax.ShapeDtypeStruct((_NP, _H), jnp.bfloat16)]
    return pl.pallas_call(_pre_body, out_shape=out_shape)(
        x_p, p16, batch_p, t2, W_emb, b_emb, Wt1, bt1, Wt2, bt2,
        Wtp, btp, Wa, Wb, eb1l)


@functools.cache
def _build_gather(nch):
    epw = nch * _CH
    ep = epw * _NW
    mesh = plsc.VectorSubcoreMesh(core_axis_name="c", subcore_axis_name="s",
                                  num_cores=_NC, num_subcores=_NS)

    def body(*args):
        return _gather_body(nch, epw, *args)

    nb = 4
    return functools.partial(
        pl.kernel,
        out_type=[jax.ShapeDtypeStruct((ep, _H), jnp.bfloat16),
                  jax.ShapeDtypeStruct((8, ep), _f32)],
        mesh=mesh,
        scratch_types=[pltpu.VMEM((nch + nb, _CH), jnp.int32),
                       pltpu.VMEM((nch + nb, _CH), jnp.int32),
                       *([pltpu.VMEM((_CH, _H), jnp.bfloat16)] * nb),
                       *([pltpu.VMEM((_CH, _H), jnp.bfloat16)] * nb),
                       *([pltpu.VMEM((8, _CH), jnp.int32)] * nb),
                       *([pltpu.VMEM((8, _CH), _f32)] * nb),
                       *([pltpu.VMEM((8, _CH), _f32)] * nb),
                       *([pltpu.SemaphoreType.DMA] * (2 * nb))],
        compiler_params=pltpu.CompilerParams(use_tc_tiling_on_sc=False),
    )(body)


_NB = 4


def _gather_body(nch, epw, ta, tb, posp, srci, dsti,
                 pre, relp, idx_s, idx_d, *bufs):
    bA = bufs[0:_NB]
    bB = bufs[_NB:2 * _NB]
    ip = bufs[2 * _NB:3 * _NB]
    pb = bufs[3 * _NB:4 * _NB]
    rb = bufs[4 * _NB:5 * _NB]
    gs = bufs[5 * _NB:6 * _NB]
    ws = bufs[6 * _NB:7 * _NB]
    c = lax.axis_index("c")
    s = lax.axis_index("s")
    wid = s * _NC + c
    base = wid * epw
    pltpu.sync_copy(srci.at[wid], idx_s)
    pltpu.sync_copy(dsti.at[wid], idx_d)

    zero16 = jnp.zeros((16,), _f32)
    for rbb in rb:
        for r in range(4, 8):
            for k in range(8):
                rbb[r, pl.ds(k * 16, 16)] = zero16

    def comp_idx(j, ip):
        for r in range(3):
            off = r * _NP
            for k in range(8):
                sl = pl.ds(k * 16, 16)
                ip[r, sl] = idx_s[j, sl] + off
                ip[3 + r, sl] = idx_d[j, sl] + off

    def fire_g(j, bA, bB, ip, pb, sem):
        pltpu.async_copy(ta.at[idx_s.at[j]], bA, sem)
        pltpu.async_copy(tb.at[idx_d.at[j]], bB, sem)
        for r in range(6):
            pltpu.async_copy(posp.at[ip.at[r]], pb.at[r], sem)

    def wait_g(j, bA, bB, ip, pb, sem):
        pltpu.make_async_copy(ta.at[idx_s.at[j]], bA, sem).wait()
        pltpu.make_async_copy(tb.at[idx_d.at[j]], bB, sem).wait()
        for r in range(6):
            pltpu.make_async_copy(posp.at[ip.at[r]], pb.at[r], sem).wait()

    def compute(bA, bB, pb, rb):
        def row(r, carry):
            for k in range(4):
                sl = pl.ds(k * 32, 32)
                bA[r, sl] = bA[r, sl] + bB[r, sl]
            return carry
        lax.fori_loop(0, _CH, row, 0, unroll=False)
        for k in range(8):
            sl = pl.ds(k * 16, 16)
            dx = pb[0, sl] - pb[3, sl]
            dy = pb[1, sl] - pb[4, sl]
            dz = pb[2, sl] - pb[5, sl]
            rb[0, sl] = dx
            rb[1, sl] = dy
            rb[2, sl] = dz
            rb[3, sl] = dx * dx + dy * dy + dz * dz

    def fire_w(j, bA, rb, sem):
        off = base + j * _CH
        pltpu.async_copy(bA, pre.at[pl.ds(off, _CH)], sem)
        pltpu.async_copy(rb, relp.at[:, pl.ds(off, _CH)], sem)

    def wait_w(j, bA, rb, sem):
        off = base + j * _CH
        pltpu.make_async_copy(bA, pre.at[pl.ds(off, _CH)], sem).wait()
        pltpu.make_async_copy(rb, relp.at[:, pl.ds(off, _CH)], sem).wait()

    for b in range(_NB):
        comp_idx(b, ip[b])
        fire_g(b, bA[b], bB[b], ip[b], pb[b], gs[b])

    def body(i, carry):
        j0 = _NB * i
        for b in range(_NB):
            j = j0 + b
            wait_g(j, bA[b], bB[b], ip[b], pb[b], gs[b])
            compute(bA[b], bB[b], pb[b], rb[b])
            fire_w(j, bA[b], rb[b], ws[b])
        for b in range(_NB):
            j = j0 + b
            wait_w(j, bA[b], rb[b], ws[b])
            comp_idx(j + _NB, ip[b])
            fire_g(j + _NB, bA[b], bB[b], ip[b], pb[b], gs[b])
        return carry

    lax.fori_loop(0, nch // _NB, body, 0)
    for b in range(_NB):
        wait_g(nch + b, bA[b], bB[b], ip[b], pb[b], gs[b])


_BLK = 2048


def _edge_body(pre_ref, relp_ref, wd, w2, b2, cwp, cbp,
               med_out, rw_out):
    relp = relp_ref[...]
    d2 = jnp.transpose(relp[3:4])
    m = _silu(pre_ref[...].astype(_f32) + d2 * wd[...])
    m = _silu(jnp.dot(m, w2[...]) + b2[...])
    cwv = (jnp.dot(m, cwp[...]) + cbp[...])[:, :1]
    med_out[...] = m
    cwr = jnp.transpose(cwv)
    rw_out[...] = jnp.concatenate(
        [relp[:3] * cwr, jnp.ones((1, _BLK), _f32),
         jnp.zeros((4, _BLK), _f32)], axis=0)


def _run_edge(pre, relp, wd, w2, b2, cwp, cbp):
    ep = pre.shape[0]
    grid = (ep // _BLK,)
    big = pl.BlockSpec((_BLK, _H), lambda i: (i, 0))
    pln = pl.BlockSpec((8, _BLK), lambda i: (0, i))
    w_spec = lambda shp: pl.BlockSpec(shp, lambda i: (0, 0))
    return pl.pallas_call(
        _edge_body,
        grid=grid,
        in_specs=[big, pln,
                  w_spec((1, _H)), w_spec((_H, _H)), w_spec((1, _H)),
                  w_spec((_H, 8)), w_spec((1, 8))],
        out_specs=[big, pln],
        out_shape=[jax.ShapeDtypeStruct((ep, _H), _f32),
                   jax.ShapeDtypeStruct((8, ep), _f32)],
    )(pre, relp, wd, w2, b2, cwp, cbp)


@functools.cache
def _build_scatter(nch):
    epw = nch * _CH
    mesh = plsc.VectorSubcoreMesh(core_axis_name="c", subcore_axis_name="s",
                                  num_cores=_NC, num_subcores=_NS)

    def body(*args):
        return _scatter_body(nch, epw, *args)

    return functools.partial(
        pl.kernel,
        out_type=[jax.ShapeDtypeStruct((_NC, _NP, _H), _f32),
                  jax.ShapeDtypeStruct((_NC, 4, _NP), _f32)],
        mesh=mesh,
        scratch_types=[pltpu.VMEM_SHARED((_NP, _H), _f32),
                       pltpu.VMEM_SHARED((_NP,), _f32),
                       pltpu.VMEM_SHARED((_NP,), _f32),
                       pltpu.VMEM_SHARED((_NP,), _f32),
                       pltpu.VMEM_SHARED((_NP,), _f32),
                       pltpu.VMEM((_CH, _H), _f32),
                       pltpu.VMEM((_CH, _H), _f32),
                       pltpu.VMEM((8, _CH), _f32),
                       pltpu.VMEM((8, _CH), _f32),
                       pltpu.VMEM((1, _CH), jnp.int32),
                       pltpu.VMEM((1, _CH), jnp.int32),
                       pltpu.SemaphoreType.DMA,
                       pltpu.SemaphoreType.DMA],
        compiler_params=pltpu.CompilerParams(use_tc_tiling_on_sc=False),
    )(body)


def _scatter_body(nch, epw, med, rw8, dsti, z128, z1,
                  pm, pr,
                  accm, accx, accy, accz, accc,
                  bm0, bm1, b80, b81, ib0, ib1, rs0, rs1):
    c = lax.axis_index("c")
    s = lax.axis_index("s")
    wid = s * _NC + c
    base = wid * epw
    r0 = s * _RT
    pltpu.sync_copy(z128.at[pl.ds(r0, _RT)], accm.at[pl.ds(r0, _RT)])
    for acc in (accx, accy, accz, accc):
        pltpu.sync_copy(z1.at[pl.ds(r0, _RT)], acc.at[pl.ds(r0, _RT)])
    plsc.subcore_barrier()

    def fire_r(j, bm, b8, ib, sem):
        jc = jnp.minimum(j, nch - 1)
        off = base + jc * _CH
        pltpu.async_copy(med.at[pl.ds(off, _CH)], bm, sem)
        pltpu.async_copy(rw8.at[:, pl.ds(off, _CH)], b8, sem)
        pltpu.async_copy(dsti.at[wid, jc], ib.at[0], sem)

    def wait_r(j, bm, b8, ib, sem):
        jc = jnp.minimum(j, nch - 1)
        off = base + jc * _CH
        pltpu.make_async_copy(med.at[pl.ds(off, _CH)], bm, sem).wait()
        pltpu.make_async_copy(rw8.at[:, pl.ds(off, _CH)], b8, sem).wait()
        pltpu.make_async_copy(dsti.at[wid, jc], ib.at[0], sem).wait()

    def adds(bm, b8, ib):
        pltpu.sync_copy(bm, accm.at[ib.at[0]], add=True)
        pltpu.sync_copy(b8.at[0], accx.at[ib.at[0]], add=True)
        pltpu.sync_copy(b8.at[1], accy.at[ib.at[0]], add=True)
        pltpu.sync_copy(b8.at[2], accz.at[ib.at[0]], add=True)
        pltpu.sync_copy(b8.at[3], accc.at[ib.at[0]], add=True)

    fire_r(0, bm0, b80, ib0, rs0)
    fire_r(1, bm1, b81, ib1, rs1)

    def body(i, carry):
        j0 = 2 * i
        j1 = j0 + 1
        wait_r(j0, bm0, b80, ib0, rs0)
        adds(bm0, b80, ib0)
        fire_r(j0 + 2, bm0, b80, ib0, rs0)
        wait_r(j1, bm1, b81, ib1, rs1)
        adds(bm1, b81, ib1)
        fire_r(j1 + 2, bm1, b81, ib1, rs1)
        return carry

    lax.fori_loop(0, nch // 2, body, 0)
    wait_r(nch, bm0, b80, ib0, rs0)
    wait_r(nch + 1, bm1, b81, ib1, rs1)
    plsc.subcore_barrier()
    pltpu.sync_copy(accm.at[pl.ds(r0, _RT)], pm.at[c, pl.ds(r0, _RT)])
    pltpu.sync_copy(accx.at[pl.ds(r0, _RT)], pr.at[c, 0, pl.ds(r0, _RT)])
    pltpu.sync_copy(accy.at[pl.ds(r0, _RT)], pr.at[c, 1, pl.ds(r0, _RT)])
    pltpu.sync_copy(accz.at[pl.ds(r0, _RT)], pr.at[c, 2, pl.ds(r0, _RT)])
    pltpu.sync_copy(accc.at[pl.ds(r0, _RT)], pr.at[c, 3, pl.ds(r0, _RT)])


def _node_body(h_ref, pc_ref, pm_ref, pr_ref,
               nW1a, nW1b, nb1, nW2, nb2, WaN, WbN, eb1N,
               h_out, pc_out, posp_out, ta_out, tb_out):
    h = h_ref[...]
    pma = pm_ref[...]
    m_agg = pma[0] + pma[1]
    pra = pr_ref[...]
    aggr = pra[0] + pra[1]
    aggT = jnp.transpose(aggr)
    cnt = jnp.maximum(aggT[:, 3:4], 1.0)
    pc = pc_ref[...] + jnp.concatenate(
        [aggT[:, :3] / cnt, jnp.zeros((_NP, 13), _f32)], axis=1)
    upd = jnp.dot(_silu(jnp.dot(h, nW1a[...]) + jnp.dot(m_agg, nW1b[...])
                        + nb1[...]), nW2[...]) + nb2[...]
    hn = h + upd
    h_out[...] = hn
    pc_out[...] = pc
    posp_out[...] = jnp.transpose(pc)
    ta_out[...] = jnp.dot(hn, WaN[...]).astype(jnp.bfloat16)
    tb_out[...] = (jnp.dot(hn, WbN[...]) + eb1N[...]).astype(jnp.bfloat16)


def _run_node(h, pc, pm, pr, nW1a, nW1b, nb1, nW2, nb2,
              WaN, WbN, eb1N):
    out_shape = [jax.ShapeDtypeStruct((_NP, _H), _f32),
                 jax.ShapeDtypeStruct((_NP, 16), _f32),
                 jax.ShapeDtypeStruct((16, _NP), _f32),
                 jax.ShapeDtypeStruct((_NP, _H), jnp.bfloat16),
                 jax.ShapeDtypeStruct((_NP, _H), jnp.bfloat16)]
    return pl.pallas_call(_node_body, out_shape=out_shape)(
        h, pc, pm, pr, nW1a, nW1b, nb1, nW2, nb2,
        WaN, WbN, eb1N)


def _out_body(h_ref, oW1, ob1, oW2, ob2, oW3, ob3, out_ref):
    o = _silu(jnp.dot(h_ref[...], oW1[...]) + ob1[...])
    o = _silu(jnp.dot(o, oW2[...]) + ob2[...])
    out_ref[...] = jnp.dot(o, oW3[...]) + ob3[...]


def _run_out(h, oW1, ob1, oW2, ob2, oW3, ob3):
    return pl.pallas_call(
        _out_body,
        out_shape=jax.ShapeDtypeStruct((_NP, 8), _f32),
    )(h, oW1, ob1, oW2, ob2, oW3, ob3)


def kernel(x, pos, edge_index, batch, t,
           W_emb, b_emb, Wt1, bt1, Wt2, bt2, Wtp, btp,
           eW1, eb1, eW2, eb2, cW, cb, nW1, nb1, nW2, nb2,
           oW1, ob1, oW2, ob2, oW3, ob3):
    x_p = jnp.zeros((_NP, 8), _f32).at[:_N].set(x.astype(_f32))
    p16 = jnp.zeros((_NP, 16), _f32).at[:_N, :3].set(pos.astype(_f32))
    batch_p = jnp.full((_NP, 1), -1, jnp.int32).at[:_N, 0].set(
        batch.astype(jnp.int32))
    t2 = t.astype(jnp.int32).reshape(_B, 1)
    pad4 = jnp.full((_NW, _NB, _CH), _N, jnp.int32)
    srci = jnp.concatenate([
        jnp.full((_EP,), _N, jnp.int32).at[:_E].set(
            edge_index[0].astype(jnp.int32)).reshape(_NW, _NCH, _CH),
        pad4], axis=1)
    dsti = jnp.concatenate([
        jnp.full((_EP,), _N, jnp.int32).at[:_E].set(
            edge_index[1].astype(jnp.int32)).reshape(_NW, _NCH, _CH),
        pad4], axis=1)
    z128 = jnp.zeros((_NP, _H), _f32)
    z1 = jnp.zeros((_NP,), _f32)

    Wa = [eW1[l][:_H] for l in range(_L)]
    Wb = [eW1[l][_H:2 * _H] for l in range(_L)]
    wd = [eW1[l][2 * _H:2 * _H + 1] for l in range(_L)]
    eb1l = [eb1[l].reshape(1, _H) for l in range(_L)]
    eb2l = [eb2[l].reshape(1, _H) for l in range(_L)]
    cwp = [jnp.pad(cW[l], ((0, 0), (0, 7))) for l in range(_L)]
    cbp = [jnp.pad(cb[l].reshape(1, 1), ((0, 0), (0, 7))) for l in range(_L)]
    nW1a = [nW1[l][:_H] for l in range(_L)]
    nW1b = [nW1[l][_H:] for l in range(_L)]
    nb1l = [nb1[l].reshape(1, _H) for l in range(_L)]
    nb2l = [nb2[l].reshape(1, _H) for l in range(_L)]

    h, pc, posp, ta, tb = _run_pre(
        x_p, p16, batch_p, t2,
        W_emb.astype(_f32), b_emb.reshape(1, _H).astype(_f32),
        Wt1.astype(_f32), bt1.reshape(1, 4 * _H).astype(_f32),
        Wt2.astype(_f32), bt2.reshape(1, 4 * _H).astype(_f32),
        Wtp.astype(_f32), btp.reshape(1, _H).astype(_f32),
        Wa[0], Wb[0], eb1l[0])

    for l in range(_L):
        pre, relp = _build_gather(_NCH)(ta, tb, posp.reshape(16 * _NP),
                                        srci, dsti)
        med, rw8 = _run_edge(pre, relp,
                             wd[l], eW2[l], eb2l[l], cwp[l], cbp[l])
        pm, pr = _build_scatter(_NCH)(med, rw8, dsti, z128, z1)
        ln = min(l + 1, _L - 1)
        h, pc, posp, ta, tb = _run_node(
            h, pc, pm, pr,
            nW1a[l], nW1b[l], nb1l[l], nW2[l], nb2l[l],
            Wa[ln], Wb[ln], eb1l[ln])

    out8 = _run_out(h, oW1.astype(_f32), ob1.reshape(1, _H).astype(_f32),
                    oW2.astype(_f32), ob2.reshape(1, _H // 2).astype(_f32),
                    oW3.astype(_f32), ob3.reshape(1, 8).astype(_f32))
    return jnp.concatenate([out8[:_N], pc[:_N, :3]], axis=-1)


# speedup vs baseline: 1.1718x; 1.1718x over previous
import functools
import numpy as np
import jax
import jax.numpy as jnp
from jax import lax
from jax.experimental import pallas as pl
from jax.experimental.pallas import tpu as pltpu
from jax.experimental.pallas import tpu_sc as plsc

_H = 128
_N = 10000
_E = 320000
_B = 32
_L = 2

_NP = 10240
_NC = 2
_NS = 16
_NW = _NC * _NS
_CH = 128
_NCH = 80
_EPW = _CH * _NCH
_EP = _EPW * _NW
_RT = _NP // _NS

_f32 = jnp.float32


def _silu(v):
    return v * jax.nn.sigmoid(v)


def _pre_body(x_ref, p16_ref, batch_ref, t_ref,
              W_emb, b_emb, Wt1, bt1, Wt2, bt2, Wtp, btp,
              Wa, Wb, eb1l,
              h_out, pc_out, posp_out, ta_out, tb_out):
    p16 = p16_ref[...]
    batch = batch_ref[...]
    onehot = (batch == lax.broadcasted_iota(jnp.int32, (_NP, _B), 1)
              ).astype(_f32)
    sums = lax.dot_general(onehot, p16, (((0,), (0,)), ((), ())))
    cnts = jnp.sum(onehot, axis=0, keepdims=True)
    mean = sums / jnp.maximum(cnts.T, 1.0)
    pos_c = p16 - jnp.dot(onehot, mean)

    tf = t_ref[...].astype(_f32)
    half = _H // 2
    freqs = jnp.exp(lax.broadcasted_iota(jnp.int32, (1, half), 1).astype(_f32)
                    * (-np.log(10000.0) / half))
    args = tf * freqs
    temb = jnp.concatenate([jnp.sin(args), jnp.cos(args)], axis=-1)
    temb = _silu(jnp.dot(temb, Wt1[...]) + bt1[...])
    temb = jnp.dot(temb, Wt2[...]) + bt2[...]
    tn = jnp.dot(_silu(temb), Wtp[...]) + btp[...]

    h = jnp.dot(x_ref[...], W_emb[...]) + b_emb[...] + jnp.dot(onehot, tn)
    h_out[...] = h
    pc_out[...] = pos_c
    posp_out[...] = jnp.transpose(pos_c)
    ta_out[...] = jnp.dot(h, Wa[...]).astype(jnp.bfloat16)
    tb_out[...] = (jnp.dot(h, Wb[...]) + eb1l[...]).astype(jnp.bfloat16)


def _run_pre(x_p, p16, batch_p, t2, W_emb, b_emb, Wt1, bt1, Wt2, bt2,
             Wtp, btp, Wa, Wb, eb1l):
    out_shape = [jax.ShapeDtypeStruct((_NP, _H), _f32),
                 jax.ShapeDtypeStruct((_NP, 16), _f32),
                 jax.ShapeDtypeStruct((16, _NP), _f32),
                 jax.ShapeDtypeStruct((_NP, _H), jnp.bfloat16),
                 jax.ShapeDtypeStruct((_NP, _H), jnp.bfloat16)]
    return pl.pallas_call(_pre_body, out_shape=out_shape)(
        x_p, p16, batch_p, t2, W_emb, b_emb, Wt1, bt1, Wt2, bt2,
        Wtp, btp, Wa, Wb, eb1l)


@functools.cache
def _build_gather(nch):
    epw = nch * _CH
    ep = epw * _NW
    mesh = plsc.VectorSubcoreMesh(core_axis_name="c", subcore_axis_name="s",
                                  num_cores=_NC, num_subcores=_NS)

    def body(*args):
        return _gather_body(nch, epw, *args)

    nb = 2
    return functools.partial(
        pl.kernel,
        out_type=[jax.ShapeDtypeStruct((ep, _H), jnp.bfloat16),
                  jax.ShapeDtypeStruct((8, ep), _f32)],
        mesh=mesh,
        scratch_types=[pltpu.VMEM((nch + nb, _CH), jnp.int32),
                       pltpu.VMEM((nch + nb, _CH), jnp.int32),
                       *([pltpu.VMEM((_CH, _H), jnp.bfloat16)] * nb),
                       *([pltpu.VMEM((_CH, _H), jnp.bfloat16)] * nb),
                       *([pltpu.VMEM((8, _CH), jnp.int32)] * nb),
                       *([pltpu.VMEM((8, _CH), _f32)] * nb),
                       *([pltpu.VMEM((8, _CH), _f32)] * nb),
                       *([pltpu.SemaphoreType.DMA] * (2 * nb))],
        compiler_params=pltpu.CompilerParams(use_tc_tiling_on_sc=False),
    )(body)


_NB = 2


def _gather_body(nch, epw, ta, tb, posp, srci, dsti,
                 pre, relp, idx_s, idx_d, *bufs):
    bA = bufs[0:_NB]
    bB = bufs[_NB:2 * _NB]
    ip = bufs[2 * _NB:3 * _NB]
    pb = bufs[3 * _NB:4 * _NB]
    rb = bufs[4 * _NB:5 * _NB]
    gs = bufs[5 * _NB:6 * _NB]
    ws = bufs[6 * _NB:7 * _NB]
    c = lax.axis_index("c")
    s = lax.axis_index("s")
    wid = s * _NC + c
    base = wid * epw
    pltpu.sync_copy(srci.at[wid], idx_s)
    pltpu.sync_copy(dsti.at[wid], idx_d)

    zero16 = jnp.zeros((16,), _f32)
    for rbb in rb:
        for r in range(4, 8):
            for k in range(8):
                rbb[r, pl.ds(k * 16, 16)] = zero16

    def comp_idx(j, ip):
        for r in range(3):
            off = r * _NP
            for k in range(8):
                sl = pl.ds(k * 16, 16)
                ip[r, sl] = idx_s[j, sl] + off
                ip[3 + r, sl] = idx_d[j, sl] + off

    def fire_g(j, bA, bB, ip, pb, sem):
        pltpu.async_copy(ta.at[idx_s.at[j]], bA, sem)
        pltpu.async_copy(tb.at[idx_d.at[j]], bB, sem)
        for r in range(6):
            pltpu.async_copy(posp.at[ip.at[r]], pb.at[r], sem)

    def wait_g(j, bA, bB, ip, pb, sem):
        pltpu.make_async_copy(ta.at[idx_s.at[j]], bA, sem).wait()
        pltpu.make_async_copy(tb.at[idx_d.at[j]], bB, sem).wait()
        for r in range(6):
            pltpu.make_async_copy(posp.at[ip.at[r]], pb.at[r], sem).wait()

    def compute(bA, bB, pb, rb):
        def row(r, carry):
            for k in range(4):
                sl = pl.ds(k * 32, 32)
                bA[r, sl] = bA[r, sl] + bB[r, sl]
            return carry
        lax.fori_loop(0, _CH, row, 0, unroll=False)
        for k in range(8):
            sl = pl.ds(k * 16, 16)
            dx = pb[0, sl] - pb[3, sl]
            dy = pb[1, sl] - pb[4, sl]
            dz = pb[2, sl] - pb[5, sl]
            rb[0, sl] = dx
            rb[1, sl] = dy
            rb[2, sl] = dz
            rb[3, sl] = dx * dx + dy * dy + dz * dz

    def fire_w(j, bA, rb, sem):
        off = base + j * _CH
        pltpu.async_copy(bA, pre.at[pl.ds(off, _CH)], sem)
        pltpu.async_copy(rb, relp.at[:, pl.ds(off, _CH)], sem)

    def wait_w(j, bA, rb, sem):
        off = base + j * _CH
        pltpu.make_async_copy(bA, pre.at[pl.ds(off, _CH)], sem).wait()
        pltpu.make_async_copy(rb, relp.at[:, pl.ds(off, _CH)], sem).wait()

    for b in range(_NB):
        comp_idx(b, ip[b])
        fire_g(b, bA[b], bB[b], ip[b], pb[b], gs[b])

    def body(i, carry):
        j0 = _NB * i
        for b in range(_NB):
            j = j0 + b
            wait_g(j, bA[b], bB[b], ip[b], pb[b], gs[b])
            compute(bA[b], bB[b], pb[b], rb[b])
            fire_w(j, bA[b], rb[b], ws[b])
        for b in range(_NB):
            j = j0 + b
            wait_w(j, bA[b], rb[b], ws[b])
            comp_idx(j + _NB, ip[b])
            fire_g(j + _NB, bA[b], bB[b], ip[b], pb[b], gs[b])
        return carry

    lax.fori_loop(0, nch // _NB, body, 0)
    for b in range(_NB):
        wait_g(nch + b, bA[b], bB[b], ip[b], pb[b], gs[b])


_BLK = 2048


def _edge_body(pre_ref, relp_ref, wd, w2, b2, cwp, cbp,
               med_out, rw_out):
    relp = relp_ref[...]
    d2 = jnp.transpose(relp[3:4])
    m = _silu(pre_ref[...].astype(_f32) + d2 * wd[...])
    m = _silu(jnp.dot(m, w2[...]) + b2[...])
    cwv = (jnp.dot(m, cwp[...]) + cbp[...])[:, :1]
    med_out[...] = m
    cwr = jnp.transpose(cwv)
    rw_out[...] = jnp.concatenate(
        [relp[:3] * cwr, jnp.ones((1, _BLK), _f32),
         jnp.zeros((4, _BLK), _f32)], axis=0)


def _run_edge(pre, relp, wd, w2, b2, cwp, cbp):
    ep = pre.shape[0]
    grid = (ep // _BLK,)
    big = pl.BlockSpec((_BLK, _H), lambda i: (i, 0))
    pln = pl.BlockSpec((8, _BLK), lambda i: (0, i))
    w_spec = lambda shp: pl.BlockSpec(shp, lambda i: (0, 0))
    return pl.pallas_call(
        _edge_body,
        grid=grid,
        in_specs=[big, pln,
                  w_spec((1, _H)), w_spec((_H, _H)), w_spec((1, _H)),
                  w_spec((_H, 8)), w_spec((1, 8))],
        out_specs=[big, pln],
        out_shape=[jax.ShapeDtypeStruct((ep, _H), _f32),
                   jax.ShapeDtypeStruct((8, ep), _f32)],
    )(pre, relp, wd, w2, b2, cwp, cbp)


@functools.cache
def _build_scatter(nch):
    epw = nch * _CH
    mesh = plsc.VectorSubcoreMesh(core_axis_name="c", subcore_axis_name="s",
                                  num_cores=_NC, num_subcores=_NS)

    def body(*args):
        return _scatter_body(nch, epw, *args)

    return functools.partial(
        pl.kernel,
        out_type=[jax.ShapeDtypeStruct((_NC, _NP, _H), _f32),
                  jax.ShapeDtypeStruct((_NC, 4, _NP), _f32)],
        mesh=mesh,
        scratch_types=[pltpu.VMEM_SHARED((_NP, _H), _f32),
                       pltpu.VMEM_SHARED((_NP,), _f32),
                       pltpu.VMEM_SHARED((_NP,), _f32),
                       pltpu.VMEM_SHARED((_NP,), _f32),
                       pltpu.VMEM_SHARED((_NP,), _f32),
                       pltpu.VMEM((_CH, _H), _f32),
                       pltpu.VMEM((_CH, _H), _f32),
                       pltpu.VMEM((8, _CH), _f32),
                       pltpu.VMEM((8, _CH), _f32),
                       pltpu.VMEM((1, _CH), jnp.int32),
                       pltpu.VMEM((1, _CH), jnp.int32),
                       pltpu.SemaphoreType.DMA,
                       pltpu.SemaphoreType.DMA],
        compiler_params=pltpu.CompilerParams(use_tc_tiling_on_sc=False),
    )(body)


def _scatter_body(nch, epw, med, rw8, dsti, z128, z1,
                  pm, pr,
                  accm, accx, accy, accz, accc,
                  bm0, bm1, b80, b81, ib0, ib1, rs0, rs1):
    c = lax.axis_index("c")
    s = lax.axis_index("s")
    wid = s * _NC + c
    base = wid * epw
    r0 = s * _RT
    pltpu.sync_copy(z128.at[pl.ds(r0, _RT)], accm.at[pl.ds(r0, _RT)])
    for acc in (accx, accy, accz, accc):
        pltpu.sync_copy(z1.at[pl.ds(r0, _RT)], acc.at[pl.ds(r0, _RT)])
    plsc.subcore_barrier()

    def fire_r(j, bm, b8, ib, sem):
        jc = jnp.minimum(j, nch - 1)
        off = base + jc * _CH
        pltpu.async_copy(med.at[pl.ds(off, _CH)], bm, sem)
        pltpu.async_copy(rw8.at[:, pl.ds(off, _CH)], b8, sem)
        pltpu.async_copy(dsti.at[wid, jc], ib.at[0], sem)

    def wait_r(j, bm, b8, ib, sem):
        jc = jnp.minimum(j, nch - 1)
        off = base + jc * _CH
        pltpu.make_async_copy(med.at[pl.ds(off, _CH)], bm, sem).wait()
        pltpu.make_async_copy(rw8.at[:, pl.ds(off, _CH)], b8, sem).wait()
        pltpu.make_async_copy(dsti.at[wid, jc], ib.at[0], sem).wait()

    def adds(bm, b8, ib):
        pltpu.sync_copy(bm, accm.at[ib.at[0]], add=True)
        pltpu.sync_copy(b8.at[0], accx.at[ib.at[0]], add=True)
        pltpu.sync_copy(b8.at[1], accy.at[ib.at[0]], add=True)
        pltpu.sync_copy(b8.at[2], accz.at[ib.at[0]], add=True)
        pltpu.sync_copy(b8.at[3], accc.at[ib.at[0]], add=True)

    fire_r(0, bm0, b80, ib0, rs0)
    fire_r(1, bm1, b81, ib1, rs1)

    def body(i, carry):
        j0 = 2 * i
        j1 = j0 + 1
        wait_r(j0, bm0, b80, ib0, rs0)
        adds(bm0, b80, ib0)
        fire_r(j0 + 2, bm0, b80, ib0, rs0)
        wait_r(j1, bm1, b81, ib1, rs1)
        adds(bm1, b81, ib1)
        fire_r(j1 + 2, bm1, b81, ib1, rs1)
        return carry

    lax.fori_loop(0, nch // 2, body, 0)
    wait_r(nch, bm0, b80, ib0, rs0)
    wait_r(nch + 1, bm1, b81, ib1, rs1)
    plsc.subcore_barrier()
    pltpu.sync_copy(accm.at[pl.ds(r0, _RT)], pm.at[c, pl.ds(r0, _RT)])
    pltpu.sync_copy(accx.at[pl.ds(r0, _RT)], pr.at[c, 0, pl.ds(r0, _RT)])
    pltpu.sync_copy(accy.at[pl.ds(r0, _RT)], pr.at[c, 1, pl.ds(r0, _RT)])
    pltpu.sync_copy(accz.at[pl.ds(r0, _RT)], pr.at[c, 2, pl.ds(r0, _RT)])
    pltpu.sync_copy(accc.at[pl.ds(r0, _RT)], pr.at[c, 3, pl.ds(r0, _RT)])


def _node_body(h_ref, pc_ref, pm_ref, pr_ref,
               nW1a, nW1b, nb1, nW2, nb2, WaN, WbN, eb1N,
               h_out, pc_out, posp_out, ta_out, tb_out):
    h = h_ref[...]
    pma = pm_ref[...]
    m_agg = pma[0] + pma[1]
    pra = pr_ref[...]
    aggr = pra[0] + pra[1]
    aggT = jnp.transpose(aggr)
    cnt = jnp.maximum(aggT[:, 3:4], 1.0)
    pc = pc_ref[...] + jnp.concatenate(
        [aggT[:, :3] / cnt, jnp.zeros((_NP, 13), _f32)], axis=1)
    upd = jnp.dot(_silu(jnp.dot(h, nW1a[...]) + jnp.dot(m_agg, nW1b[...])
                        + nb1[...]), nW2[...]) + nb2[...]
    hn = h + upd
    h_out[...] = hn
    pc_out[...] = pc
    posp_out[...] = jnp.transpose(pc)
    ta_out[...] = jnp.dot(hn, WaN[...]).astype(jnp.bfloat16)
    tb_out[...] = (jnp.dot(hn, WbN[...]) + eb1N[...]).astype(jnp.bfloat16)


def _run_node(h, pc, pm, pr, nW1a, nW1b, nb1, nW2, nb2,
              WaN, WbN, eb1N):
    out_shape = [jax.ShapeDtypeStruct((_NP, _H), _f32),
                 jax.ShapeDtypeStruct((_NP, 16), _f32),
                 jax.ShapeDtypeStruct((16, _NP), _f32),
                 jax.ShapeDtypeStruct((_NP, _H), jnp.bfloat16),
                 jax.ShapeDtypeStruct((_NP, _H), jnp.bfloat16)]
    return pl.pallas_call(_node_body, out_shape=out_shape)(
        h, pc, pm, pr, nW1a, nW1b, nb1, nW2, nb2,
        WaN, WbN, eb1N)


def _out_body(h_ref, oW1, ob1, oW2, ob2, oW3, ob3, out_ref):
    o = _silu(jnp.dot(h_ref[...], oW1[...]) + ob1[...])
    o = _silu(jnp.dot(o, oW2[...]) + ob2[...])
    out_ref[...] = jnp.dot(o, oW3[...]) + ob3[...]


def _run_out(h, oW1, ob1, oW2, ob2, oW3, ob3):
    return pl.pallas_call(
        _out_body,
        out_shape=jax.ShapeDtypeStruct((_NP, 8), _f32),
    )(h, oW1, ob1, oW2, ob2, oW3, ob3)


def kernel(x, pos, edge_index, batch, t,
           W_emb, b_emb, Wt1, bt1, Wt2, bt2, Wtp, btp,
           eW1, eb1, eW2, eb2, cW, cb, nW1, nb1, nW2, nb2,
           oW1, ob1, oW2, ob2, oW3, ob3):
    x_p = jnp.zeros((_NP, 8), _f32).at[:_N].set(x.astype(_f32))
    p16 = jnp.zeros((_NP, 16), _f32).at[:_N, :3].set(pos.astype(_f32))
    batch_p = jnp.full((_NP, 1), -1, jnp.int32).at[:_N, 0].set(
        batch.astype(jnp.int32))
    t2 = t.astype(jnp.int32).reshape(_B, 1)
    pad4 = jnp.full((_NW, _NB, _CH), _N, jnp.int32)
    srci = jnp.concatenate([
        jnp.full((_EP,), _N, jnp.int32).at[:_E].set(
            edge_index[0].astype(jnp.int32)).reshape(_NW, _NCH, _CH),
        pad4], axis=1)
    dsti = jnp.concatenate([
        jnp.full((_EP,), _N, jnp.int32).at[:_E].set(
            edge_index[1].astype(jnp.int32)).reshape(_NW, _NCH, _CH),
        pad4], axis=1)
    z128 = jnp.zeros((_NP, _H), _f32)
    z1 = jnp.zeros((_NP,), _f32)

    Wa = [eW1[l][:_H] for l in range(_L)]
    Wb = [eW1[l][_H:2 * _H] for l in range(_L)]
    wd = [eW1[l][2 * _H:2 * _H + 1] for l in range(_L)]
    eb1l = [eb1[l].reshape(1, _H) for l in range(_L)]
    eb2l = [eb2[l].reshape(1, _H) for l in range(_L)]
    cwp = [jnp.pad(cW[l], ((0, 0), (0, 7))) for l in range(_L)]
    cbp = [jnp.pad(cb[l].reshape(1, 1), ((0, 0), (0, 7))) for l in range(_L)]
    nW1a = [nW1[l][:_H] for l in range(_L)]
    nW1b = [nW1[l][_H:] for l in range(_L)]
    nb1l = [nb1[l].reshape(1, _H) for l in range(_L)]
    nb2l = [nb2[l].reshape(1, _H) for l in range(_L)]

    h, pc, posp, ta, tb = _run_pre(
        x_p, p16, batch_p, t2,
        W_emb.astype(_f32), b_emb.reshape(1, _H).astype(_f32),
        Wt1.astype(_f32), bt1.reshape(1, 4 * _H).astype(_f32),
        Wt2.astype(_f32), bt2.reshape(1, 4 * _H).astype(_f32),
        Wtp.astype(_f32), btp.reshape(1, _H).astype(_f32),
        Wa[0], Wb[0], eb1l[0])

    for l in range(_L):
        pre, relp = _build_gather(_NCH)(ta, tb, posp.reshape(16 * _NP),
                                        srci, dsti)
        med, rw8 = _run_edge(pre, relp,
                             wd[l], eW2[l], eb2l[l], cwp[l], cbp[l])
        pm, pr = _build_scatter(_NCH)(med, rw8, dsti, z128, z1)
        ln = min(l + 1, _L - 1)
        h, pc, posp, ta, tb = _run_node(
            h, pc, pm, pr,
            nW1a[l], nW1b[l], nb1l[l], nW2[l], nb2l[l],
            Wa[ln], Wb[ln], eb1l[ln])

    out8 = _run_out(h, oW1.astype(_f32), ob1.reshape(1, _H).astype(_f32),
                    oW2.astype(_f32), ob2.reshape(1, _H // 2).astype(_f32),
                    oW3.astype(_f32), ob3.reshape(1, 8).astype(_f32))
    return jnp.concatenate([out8[:_N], pc[:_N, :3]], axis=-1)


# speedup vs baseline: 1.2145x; 1.0364x over previous
import functools
import numpy as np
import jax
import jax.numpy as jnp
from jax import lax
from jax.experimental import pallas as pl
from jax.experimental.pallas import tpu as pltpu
from jax.experimental.pallas import tpu_sc as plsc

_H = 128
_N = 10000
_E = 320000
_B = 32
_L = 2

_NP = 10240
_NC = 2
_NS = 16
_NW = _NC * _NS
_CH = 128
_NCH = 80
_EPW = _CH * _NCH
_EP = _EPW * _NW
_RT = _NP // _NS

_f32 = jnp.float32


def _silu(v):
    return v * jax.nn.sigmoid(v)


def _pre_body(x_ref, p16_ref, batch_ref, t_ref,
              W_emb, b_emb, Wt1, bt1, Wt2, bt2, Wtp, btp,
              Wa, Wb, eb1l,
              h_out, pc_out, posp_out, ta_out, tb_out):
    p16 = p16_ref[...]
    batch = batch_ref[...]
    onehot = (batch == lax.broadcasted_iota(jnp.int32, (_NP, _B), 1)
              ).astype(_f32)
    sums = lax.dot_general(onehot, p16, (((0,), (0,)), ((), ())))
    cnts = jnp.sum(onehot, axis=0, keepdims=True)
    mean = sums / jnp.maximum(cnts.T, 1.0)
    pos_c = p16 - jnp.dot(onehot, mean)

    tf = t_ref[...].astype(_f32)
    half = _H // 2
    freqs = jnp.exp(lax.broadcasted_iota(jnp.int32, (1, half), 1).astype(_f32)
                    * (-np.log(10000.0) / half))
    args = tf * freqs
    temb = jnp.concatenate([jnp.sin(args), jnp.cos(args)], axis=-1)
    temb = _silu(jnp.dot(temb, Wt1[...]) + bt1[...])
    temb = jnp.dot(temb, Wt2[...]) + bt2[...]
    tn = jnp.dot(_silu(temb), Wtp[...]) + btp[...]

    h = jnp.dot(x_ref[...], W_emb[...]) + b_emb[...] + jnp.dot(onehot, tn)
    h_out[...] = h
    pc_out[...] = pos_c
    posp_out[...] = jnp.transpose(pos_c)
    ta_out[...] = jnp.dot(h, Wa[...]).astype(jnp.bfloat16)
    tb_out[...] = (jnp.dot(h, Wb[...]) + eb1l[...]).astype(jnp.bfloat16)


def _run_pre(x_p, p16, batch_p, t2, W_emb, b_emb, Wt1, bt1, Wt2, bt2,
             Wtp, btp, Wa, Wb, eb1l):
    out_shape = [jax.ShapeDtypeStruct((_NP, _H), _f32),
                 jax.ShapeDtypeStruct((_NP, 16), _f32),
                 jax.ShapeDtypeStruct((16, _NP), _f32),
                 jax.ShapeDtypeStruct((_NP, _H), jnp.bfloat16),
                 jax.ShapeDtypeStruct((_NP, _H), jnp.bfloat16)]
    return pl.pallas_call(_pre_body, out_shape=out_shape)(
        x_p, p16, batch_p, t2, W_emb, b_emb, Wt1, bt1, Wt2, bt2,
        Wtp, btp, Wa, Wb, eb1l)


@functools.cache
def _build_gather(nch):
    epw = nch * _CH
    ep = epw * _NW
    mesh = plsc.VectorSubcoreMesh(core_axis_name="c", subcore_axis_name="s",
                                  num_cores=_NC, num_subcores=_NS)

    def body(*args):
        return _gather_body(nch, epw, *args)

    nb = 2
    return functools.partial(
        pl.kernel,
        out_type=[jax.ShapeDtypeStruct((ep, _H), jnp.bfloat16),
                  jax.ShapeDtypeStruct((8, ep), _f32)],
        mesh=mesh,
        scratch_types=[pltpu.VMEM((nch + nb, _CH), jnp.int32),
                       pltpu.VMEM((nch + nb, _CH), jnp.int32),
                       *([pltpu.VMEM((_CH, _H), jnp.bfloat16)] * nb),
                       *([pltpu.VMEM((_CH, _H), jnp.bfloat16)] * nb),
                       *([pltpu.VMEM((8, _CH), jnp.int32)] * nb),
                       *([pltpu.VMEM((8, _CH), _f32)] * nb),
                       *([pltpu.VMEM((8, _CH), _f32)] * nb),
                       *([pltpu.SemaphoreType.DMA] * (2 * nb))],
        compiler_params=pltpu.CompilerParams(use_tc_tiling_on_sc=False),
    )(body)


_NB = 2


def _gather_body(nch, epw, ta, tb, posp, srci, dsti,
                 pre, relp, idx_s, idx_d, *bufs):
    bA = bufs[0:_NB]
    bB = bufs[_NB:2 * _NB]
    ip = bufs[2 * _NB:3 * _NB]
    pb = bufs[3 * _NB:4 * _NB]
    rb = bufs[4 * _NB:5 * _NB]
    gs = bufs[5 * _NB:6 * _NB]
    ws = bufs[6 * _NB:7 * _NB]
    c = lax.axis_index("c")
    s = lax.axis_index("s")
    wid = s * _NC + c
    base = wid * epw
    pltpu.sync_copy(srci.at[wid], idx_s)
    pltpu.sync_copy(dsti.at[wid], idx_d)

    zero16 = jnp.zeros((16,), _f32)
    for rbb in rb:
        for r in range(4, 8):
            for k in range(8):
                rbb[r, pl.ds(k * 16, 16)] = zero16

    def comp_idx(j, ip):
        for r in range(3):
            off = r * _NP
            for k in range(8):
                sl = pl.ds(k * 16, 16)
                ip[r, sl] = idx_s[j, sl] + off
                ip[3 + r, sl] = idx_d[j, sl] + off

    def fire_g(j, bA, bB, ip, pb, sem):
        pltpu.async_copy(ta.at[idx_s.at[j]], bA, sem)
        pltpu.async_copy(tb.at[idx_d.at[j]], bB, sem)
        for r in range(6):
            pltpu.async_copy(posp.at[ip.at[r]], pb.at[r], sem)

    def wait_g(j, bA, bB, ip, pb, sem):
        pltpu.make_async_copy(ta.at[idx_s.at[j]], bA, sem).wait()
        pltpu.make_async_copy(tb.at[idx_d.at[j]], bB, sem).wait()
        for r in range(6):
            pltpu.make_async_copy(posp.at[ip.at[r]], pb.at[r], sem).wait()

    def compute(bA, bB, pb, rb):
        def row(r, carry):
            for k in range(4):
                sl = pl.ds(k * 32, 32)
                bA[r, sl] = bA[r, sl] + bB[r, sl]
            return carry
        lax.fori_loop(0, _CH, row, 0, unroll=False)
        for k in range(8):
            sl = pl.ds(k * 16, 16)
            dx = pb[0, sl] - pb[3, sl]
            dy = pb[1, sl] - pb[4, sl]
            dz = pb[2, sl] - pb[5, sl]
            rb[0, sl] = dx
            rb[1, sl] = dy
            rb[2, sl] = dz
            rb[3, sl] = dx * dx + dy * dy + dz * dz

    def fire_w(j, bA, rb, sem):
        off = base + j * _CH
        pltpu.async_copy(bA, pre.at[pl.ds(off, _CH)], sem)
        pltpu.async_copy(rb, relp.at[:, pl.ds(off, _CH)], sem)

    def wait_w(j, bA, rb, sem):
        off = base + j * _CH
        pltpu.make_async_copy(bA, pre.at[pl.ds(off, _CH)], sem).wait()
        pltpu.make_async_copy(rb, relp.at[:, pl.ds(off, _CH)], sem).wait()

    for b in range(_NB):
        comp_idx(b, ip[b])
        fire_g(b, bA[b], bB[b], ip[b], pb[b], gs[b])

    def body(i, carry):
        j0 = _NB * i
        for b in range(_NB):
            j = j0 + b
            wait_g(j, bA[b], bB[b], ip[b], pb[b], gs[b])
            compute(bA[b], bB[b], pb[b], rb[b])
            fire_w(j, bA[b], rb[b], ws[b])
        for b in range(_NB):
            j = j0 + b
            wait_w(j, bA[b], rb[b], ws[b])
            comp_idx(j + _NB, ip[b])
            fire_g(j + _NB, bA[b], bB[b], ip[b], pb[b], gs[b])
        return carry

    lax.fori_loop(0, nch // _NB, body, 0)
    for b in range(_NB):
        wait_g(nch + b, bA[b], bB[b], ip[b], pb[b], gs[b])


_BLK = 4096


def _edge_body(pre_ref, relp_ref, wd, w2, b2, cwp, cbp,
               med_out, rw_out):
    relp = relp_ref[...]
    d2 = jnp.transpose(relp[3:4])
    m = _silu(pre_ref[...].astype(_f32) + d2 * wd[...])
    m = _silu(jnp.dot(m, w2[...]) + b2[...])
    cwv = (jnp.dot(m, cwp[...]) + cbp[...])[:, :1]
    med_out[...] = m
    cwr = jnp.transpose(cwv)
    rw_out[...] = jnp.concatenate(
        [relp[:3] * cwr, jnp.ones((1, _BLK), _f32),
         jnp.zeros((4, _BLK), _f32)], axis=0)


def _run_edge(pre, relp, wd, w2, b2, cwp, cbp):
    ep = pre.shape[0]
    grid = (ep // _BLK,)
    big = pl.BlockSpec((_BLK, _H), lambda i: (i, 0))
    pln = pl.BlockSpec((8, _BLK), lambda i: (0, i))
    w_spec = lambda shp: pl.BlockSpec(shp, lambda i: (0, 0))
    return pl.pallas_call(
        _edge_body,
        grid=grid,
        in_specs=[big, pln,
                  w_spec((1, _H)), w_spec((_H, _H)), w_spec((1, _H)),
                  w_spec((_H, 8)), w_spec((1, 8))],
        out_specs=[big, pln],
        out_shape=[jax.ShapeDtypeStruct((ep, _H), _f32),
                   jax.ShapeDtypeStruct((8, ep), _f32)],
    )(pre, relp, wd, w2, b2, cwp, cbp)


@functools.cache
def _build_scatter(nch):
    epw = nch * _CH
    mesh = plsc.VectorSubcoreMesh(core_axis_name="c", subcore_axis_name="s",
                                  num_cores=_NC, num_subcores=_NS)

    def body(*args):
        return _scatter_body(nch, epw, *args)

    return functools.partial(
        pl.kernel,
        out_type=[jax.ShapeDtypeStruct((_NC, _NP, _H), _f32),
                  jax.ShapeDtypeStruct((_NC, 4, _NP), _f32)],
        mesh=mesh,
        scratch_types=[pltpu.VMEM_SHARED((_NP, _H), _f32),
                       pltpu.VMEM_SHARED((_NP,), _f32),
                       pltpu.VMEM_SHARED((_NP,), _f32),
                       pltpu.VMEM_SHARED((_NP,), _f32),
                       pltpu.VMEM_SHARED((_NP,), _f32),
                       pltpu.VMEM((_CH, _H), _f32),
                       pltpu.VMEM((_CH, _H), _f32),
                       pltpu.VMEM((8, _CH), _f32),
                       pltpu.VMEM((8, _CH), _f32),
                       pltpu.VMEM((1, _CH), jnp.int32),
                       pltpu.VMEM((1, _CH), jnp.int32),
                       pltpu.SemaphoreType.DMA,
                       pltpu.SemaphoreType.DMA],
        compiler_params=pltpu.CompilerParams(use_tc_tiling_on_sc=False),
    )(body)


def _scatter_body(nch, epw, med, rw8, dsti, z128, z1,
                  pm, pr,
                  accm, accx, accy, accz, accc,
                  bm0, bm1, b80, b81, ib0, ib1, rs0, rs1):
    c = lax.axis_index("c")
    s = lax.axis_index("s")
    wid = s * _NC + c
    base = wid * epw
    r0 = s * _RT
    pltpu.sync_copy(z128.at[pl.ds(r0, _RT)], accm.at[pl.ds(r0, _RT)])
    for acc in (accx, accy, accz, accc):
        pltpu.sync_copy(z1.at[pl.ds(r0, _RT)], acc.at[pl.ds(r0, _RT)])
    plsc.subcore_barrier()

    def fire_r(j, bm, b8, ib, sem):
        jc = jnp.minimum(j, nch - 1)
        off = base + jc * _CH
        pltpu.async_copy(med.at[pl.ds(off, _CH)], bm, sem)
        pltpu.async_copy(rw8.at[:, pl.ds(off, _CH)], b8, sem)
        pltpu.async_copy(dsti.at[wid, jc], ib.at[0], sem)

    def wait_r(j, bm, b8, ib, sem):
        jc = jnp.minimum(j, nch - 1)
        off = base + jc * _CH
        pltpu.make_async_copy(med.at[pl.ds(off, _CH)], bm, sem).wait()
        pltpu.make_async_copy(rw8.at[:, pl.ds(off, _CH)], b8, sem).wait()
        pltpu.make_async_copy(dsti.at[wid, jc], ib.at[0], sem).wait()

    def adds(bm, b8, ib):
        pltpu.sync_copy(bm, accm.at[ib.at[0]], add=True)
        pltpu.sync_copy(b8.at[0], accx.at[ib.at[0]], add=True)
        pltpu.sync_copy(b8.at[1], accy.at[ib.at[0]], add=True)
        pltpu.sync_copy(b8.at[2], accz.at[ib.at[0]], add=True)
        pltpu.sync_copy(b8.at[3], accc.at[ib.at[0]], add=True)

    fire_r(0, bm0, b80, ib0, rs0)
    fire_r(1, bm1, b81, ib1, rs1)

    def body(i, carry):
        j0 = 2 * i
        j1 = j0 + 1
        wait_r(j0, bm0, b80, ib0, rs0)
        adds(bm0, b80, ib0)
        fire_r(j0 + 2, bm0, b80, ib0, rs0)
        wait_r(j1, bm1, b81, ib1, rs1)
        adds(bm1, b81, ib1)
        fire_r(j1 + 2, bm1, b81, ib1, rs1)
        return carry

    lax.fori_loop(0, nch // 2, body, 0)
    wait_r(nch, bm0, b80, ib0, rs0)
    wait_r(nch + 1, bm1, b81, ib1, rs1)
    plsc.subcore_barrier()
    pltpu.sync_copy(accm.at[pl.ds(r0, _RT)], pm.at[c, pl.ds(r0, _RT)])
    pltpu.sync_copy(accx.at[pl.ds(r0, _RT)], pr.at[c, 0, pl.ds(r0, _RT)])
    pltpu.sync_copy(accy.at[pl.ds(r0, _RT)], pr.at[c, 1, pl.ds(r0, _RT)])
    pltpu.sync_copy(accz.at[pl.ds(r0, _RT)], pr.at[c, 2, pl.ds(r0, _RT)])
    pltpu.sync_copy(accc.at[pl.ds(r0, _RT)], pr.at[c, 3, pl.ds(r0, _RT)])


def _node_body(h_ref, pc_ref, pm_ref, pr_ref,
               nW1a, nW1b, nb1, nW2, nb2, WaN, WbN, eb1N,
               h_out, pc_out, posp_out, ta_out, tb_out):
    h = h_ref[...]
    pma = pm_ref[...]
    m_agg = pma[0] + pma[1]
    pra = pr_ref[...]
    aggr = pra[0] + pra[1]
    aggT = jnp.transpose(aggr)
    cnt = jnp.maximum(aggT[:, 3:4], 1.0)
    pc = pc_ref[...] + jnp.concatenate(
        [aggT[:, :3] / cnt, jnp.zeros((_NP, 13), _f32)], axis=1)
    upd = jnp.dot(_silu(jnp.dot(h, nW1a[...]) + jnp.dot(m_agg, nW1b[...])
                        + nb1[...]), nW2[...]) + nb2[...]
    hn = h + upd
    h_out[...] = hn
    pc_out[...] = pc
    posp_out[...] = jnp.transpose(pc)
    ta_out[...] = jnp.dot(hn, WaN[...]).astype(jnp.bfloat16)
    tb_out[...] = (jnp.dot(hn, WbN[...]) + eb1N[...]).astype(jnp.bfloat16)


def _run_node(h, pc, pm, pr, nW1a, nW1b, nb1, nW2, nb2,
              WaN, WbN, eb1N):
    out_shape = [jax.ShapeDtypeStruct((_NP, _H), _f32),
                 jax.ShapeDtypeStruct((_NP, 16), _f32),
                 jax.ShapeDtypeStruct((16, _NP), _f32),
                 jax.ShapeDtypeStruct((_NP, _H), jnp.bfloat16),
                 jax.ShapeDtypeStruct((_NP, _H), jnp.bfloat16)]
    return pl.pallas_call(_node_body, out_shape=out_shape)(
        h, pc, pm, pr, nW1a, nW1b, nb1, nW2, nb2,
        WaN, WbN, eb1N)


def _out_body(h_ref, oW1, ob1, oW2, ob2, oW3, ob3, out_ref):
    o = _silu(jnp.dot(h_ref[...], oW1[...]) + ob1[...])
    o = _silu(jnp.dot(o, oW2[...]) + ob2[...])
    out_ref[...] = jnp.dot(o, oW3[...]) + ob3[...]


def _run_out(h, oW1, ob1, oW2, ob2, oW3, ob3):
    return pl.pallas_call(
        _out_body,
        out_shape=jax.ShapeDtypeStruct((_NP, 8), _f32),
    )(h, oW1, ob1, oW2, ob2, oW3, ob3)


def kernel(x, pos, edge_index, batch, t,
           W_emb, b_emb, Wt1, bt1, Wt2, bt2, Wtp, btp,
           eW1, eb1, eW2, eb2, cW, cb, nW1, nb1, nW2, nb2,
           oW1, ob1, oW2, ob2, oW3, ob3):
    x_p = jnp.zeros((_NP, 8), _f32).at[:_N].set(x.astype(_f32))
    p16 = jnp.zeros((_NP, 16), _f32).at[:_N, :3].set(pos.astype(_f32))
    batch_p = jnp.full((_NP, 1), -1, jnp.int32).at[:_N, 0].set(
        batch.astype(jnp.int32))
    t2 = t.astype(jnp.int32).reshape(_B, 1)
    pad4 = jnp.full((_NW, _NB, _CH), _N, jnp.int32)
    srci = jnp.concatenate([
        jnp.full((_EP,), _N, jnp.int32).at[:_E].set(
            edge_index[0].astype(jnp.int32)).reshape(_NW, _NCH, _CH),
        pad4], axis=1)
    dsti = jnp.concatenate([
        jnp.full((_EP,), _N, jnp.int32).at[:_E].set(
            edge_index[1].astype(jnp.int32)).reshape(_NW, _NCH, _CH),
        pad4], axis=1)
    z128 = jnp.zeros((_NP, _H), _f32)
    z1 = jnp.zeros((_NP,), _f32)

    Wa = [eW1[l][:_H] for l in range(_L)]
    Wb = [eW1[l][_H:2 * _H] for l in range(_L)]
    wd = [eW1[l][2 * _H:2 * _H + 1] for l in range(_L)]
    eb1l = [eb1[l].reshape(1, _H) for l in range(_L)]
    eb2l = [eb2[l].reshape(1, _H) for l in range(_L)]
    cwp = [jnp.pad(cW[l], ((0, 0), (0, 7))) for l in range(_L)]
    cbp = [jnp.pad(cb[l].reshape(1, 1), ((0, 0), (0, 7))) for l in range(_L)]
    nW1a = [nW1[l][:_H] for l in range(_L)]
    nW1b = [nW1[l][_H:] for l in range(_L)]
    nb1l = [nb1[l].reshape(1, _H) for l in range(_L)]
    nb2l = [nb2[l].reshape(1, _H) for l in range(_L)]

    h, pc, posp, ta, tb = _run_pre(
        x_p, p16, batch_p, t2,
        W_emb.astype(_f32), b_emb.reshape(1, _H).astype(_f32),
        Wt1.astype(_f32), bt1.reshape(1, 4 * _H).astype(_f32),
        Wt2.astype(_f32), bt2.reshape(1, 4 * _H).astype(_f32),
        Wtp.astype(_f32), btp.reshape(1, _H).astype(_f32),
        Wa[0], Wb[0], eb1l[0])

    for l in range(_L):
        pre, relp = _build_gather(_NCH)(ta, tb, posp.reshape(16 * _NP),
                                        srci, dsti)
        med, rw8 = _run_edge(pre, relp,
                             wd[l], eW2[l], eb2l[l], cwp[l], cbp[l])
        pm, pr = _build_scatter(_NCH)(med, rw8, dsti, z128, z1)
        ln = min(l + 1, _L - 1)
        h, pc, posp, ta, tb = _run_node(
            h, pc, pm, pr,
            nW1a[l], nW1b[l], nb1l[l], nW2[l], nb2l[l],
            Wa[ln], Wb[ln], eb1l[ln])

    out8 = _run_out(h, oW1.astype(_f32), ob1.reshape(1, _H).astype(_f32),
                    oW2.astype(_f32), ob2.reshape(1, _H // 2).astype(_f32),
                    oW3.astype(_f32), ob3.reshape(1, 8).astype(_f32))
    return jnp.concatenate([out8[:_N], pc[:_N, :3]], axis=-1)
